# baseline (speedup 1.0000x reference)
import functools

import jax
import jax.numpy as jnp
from jax import lax
from jax.experimental import pallas as pl
from jax.experimental.pallas import tpu as pltpu
from jax.experimental.pallas import tpu_sc as plsc

L = 64
M = 64
BATCH = 4096
NUP = 33
KK = NUP * NUP
ROWS = 2 * L * NUP
S_STRIDE = L * NUP
I_STRIDE = NUP
NU_STRIDE = 1

_NC, _NS, _NL = 2, 16, 16
NW = _NC * _NS
SPW = BATCH // NW
GRP = SPW // 16


def _table_body(eps_ref, tab_ref):
    i = pl.program_id(1)
    e = eps_ref[...].reshape(M, NUP, NUP)
    nu = lax.broadcasted_iota(jnp.int32, (NUP, NUP), 0)
    nd = lax.broadcasted_iota(jnp.int32, (NUP, NUP), 1)
    mask = (nd == i - nu).astype(jnp.float32)
    b = jnp.sum(e * mask[None], axis=2)
    r = lax.broadcasted_iota(jnp.int32, (M, M), 0)
    c = lax.broadcasted_iota(jnp.int32, (M, M), 1)
    eye = (r == c).astype(jnp.float32)
    y = lax.dot_general(b, eye, (((0,), (0,)), ((), ())),
                        precision=jax.lax.Precision.HIGHEST,
                        preferred_element_type=jnp.float32)
    tab_ref[...] = y.reshape(1, 1, NUP, M)


def _idx_body(in_ref, idx_ref):
    s = in_ref[...].astype(jnp.float32)
    row = lax.broadcasted_iota(jnp.int32, (L, L), 0)
    col = lax.broadcasted_iota(jnp.int32, (L, L), 1)
    tri = (row < col).astype(jnp.float32)
    nu = jax.lax.dot(s, tri, precision=jax.lax.Precision.HIGHEST)
    site = lax.broadcasted_iota(jnp.int32, (BATCH, L), 1).astype(jnp.float32)
    idx = s * S_STRIDE + site * I_STRIDE + nu * NU_STRIDE
    idx_ref[...] = idx.astype(jnp.int32)


PAIRW = 2 * L
NPAIR = SPW // 2


def _sc_body(table_hbm, idx_hbm, out_hbm, idx_v, rows0, rows1, tmp_v, out_v,
             sem0, sem1):
    wid = lax.axis_index("s") * _NC + lax.axis_index("c")
    base = wid * SPW * L
    pltpu.sync_copy(idx_hbm.at[pl.ds(base, SPW * L)], idx_v)
    iota = lax.iota(jnp.int32, 16)

    def product(rows_v, off):
        def prod(j, accs):
            accs = list(accs)
            for r in range(8):
                row = off + 8 * j + r
                c = (r % 2) * 4
                for k in range(4):
                    accs[c + k] = accs[c + k] * rows_v[row, pl.ds(16 * k, 16)]
            return tuple(accs)

        ones = jnp.ones((16,), jnp.float32)
        accs = lax.fori_loop(0, L // 8, prod, (ones,) * 8)
        return (accs[0] * accs[4] + accs[1] * accs[5]
                + accs[2] * accs[6] + accs[3] * accs[7])

    def gather_pair(p, dst, sem):
        return pltpu.async_copy(
            table_hbm.at[idx_v.at[pl.ds(p * PAIRW, PAIRW)]], dst, sem)

    def wait_pair(p, dst, sem):
        pltpu.make_async_copy(
            table_hbm.at[idx_v.at[pl.ds(p * PAIRW, PAIRW)]], dst, sem).wait()

    gather_pair(0, rows0, sem0)

    def group(g, _):
        def quad(qq, _):
            p0 = g * 8 + 2 * qq
            s0 = 4 * qq
            gather_pair(p0 + 1, rows1, sem1)
            wait_pair(p0, rows0, sem0)
            tot_a = product(rows0, 0)
            tot_b = product(rows0, L)

            @pl.when(p0 < NPAIR - 2)
            def _():
                gather_pair(p0 + 2, rows0, sem0)

            wait_pair(p0 + 1, rows1, sem1)
            tot_c = product(rows1, 0)
            tot_d = product(rows1, L)
            tmp_v[pl.ds(s0 * 16, 16)] = tot_a
            tmp_v[pl.ds((s0 + 1) * 16, 16)] = tot_b
            tmp_v[pl.ds((s0 + 2) * 16, 16)] = tot_c
            tmp_v[pl.ds((s0 + 3) * 16, 16)] = tot_d
            return 0

        lax.fori_loop(0, 4, quad, 0)
        acc = jnp.zeros((16,), jnp.float32)
        for j in range(16):
            acc = acc + plsc.load_gather(tmp_v, [iota * 16 + j])
        out_v[pl.ds(g * 16, 16)] = acc
        return 0

    lax.fori_loop(0, GRP, group, 0)
    pltpu.sync_copy(out_v, out_hbm.at[pl.ds(wid * SPW, SPW)])


@jax.jit
def _seg_gps(epsilon, inputs_i32):
    table = pl.pallas_call(
        _table_body,
        grid=(2, L),
        in_specs=[
            pl.BlockSpec((1, M, 1, NUP, NUP), lambda s, i: (s, 0, i, 0, 0))],
        out_specs=pl.BlockSpec((1, 1, NUP, M), lambda s, i: (s, i, 0, 0)),
        out_shape=jax.ShapeDtypeStruct((2, L, NUP, M), jnp.float32),
    )(epsilon)
    idx = pl.pallas_call(
        _idx_body,
        out_shape=jax.ShapeDtypeStruct((BATCH, L), jnp.int32),
    )(inputs_i32)
    mesh = plsc.VectorSubcoreMesh(core_axis_name="c", subcore_axis_name="s")
    return pl.kernel(
        _sc_body,
        mesh=mesh,
        compiler_params=pltpu.CompilerParams(
            needs_layout_passes=False, use_tc_tiling_on_sc=False),
        out_type=jax.ShapeDtypeStruct((BATCH,), jnp.float32),
        scratch_types=[
            pltpu.VMEM((SPW * L,), jnp.int32),
            pltpu.VMEM((PAIRW, M), jnp.float32),
            pltpu.VMEM((PAIRW, M), jnp.float32),
            pltpu.VMEM((256,), jnp.float32),
            pltpu.VMEM((SPW,), jnp.float32),
            pltpu.SemaphoreType.DMA,
            pltpu.SemaphoreType.DMA,
        ],
    )(table.reshape(ROWS, M), idx.reshape(-1))


def kernel(inputs, epsilon):
    return _seg_gps(epsilon, inputs.astype(jnp.int32))


# speedup vs baseline: 1.1303x; 1.1303x over previous
import functools

import jax
import jax.numpy as jnp
from jax import lax
from jax.experimental import pallas as pl
from jax.experimental.pallas import tpu as pltpu
from jax.experimental.pallas import tpu_sc as plsc

L = 64
M = 64
BATCH = 4096
NUP = 33
TROWS = 2 * L * NUP
S_STRIDE = L * NUP
I_STRIDE = NUP
ES = M * L * NUP * NUP
EM = L * NUP * NUP
EI = NUP * NUP + 1
ENU = NUP - 1

_NC, _NS = 2, 16
NW = _NC * _NS
SPW = BATCH // NW
GRP = SPW // 16
PAIRW = 2 * L
NPAIR = SPW // 2


def _sc_body(eps_hbm, inputs_hbm, out_hbm, tab, in_v, idx_eb, ebuf, idx_v,
             rows0, rows1, tmp_v, out_v, semA, sem0, sem1):
    sub = lax.axis_index("s")
    core = lax.axis_index("c")
    wid = sub * _NC + core
    rowcore = core * TROWS
    iota = lax.iota(jnp.int32, 16)

    def build_pair(q, _):
        pp = sub * 8 + q
        sbit = pp // L
        i = pp - sbit * L
        ebase = sbit * ES + EI * i

        def nu_idx(nu, _):
            b = ebase + ENU * nu
            for mc in range(4):
                idx_eb[pl.ds(nu * M + mc * 16, 16)] = (
                    b + EM * (iota + 16 * mc))
            return 0

        lax.fori_loop(0, NUP, nu_idx, 0)
        for nu in range(NUP):
            pltpu.async_copy(
                eps_hbm.at[idx_eb.at[pl.ds(nu * M, M)]], ebuf.at[nu], semA)
        for nu in range(NUP):
            pltpu.make_async_copy(
                eps_hbm.at[idx_eb.at[pl.ds(nu * M, M)]], ebuf.at[nu],
                semA).wait()
        rowbase = rowcore + sbit * S_STRIDE + i * I_STRIDE
        pltpu.sync_copy(ebuf, tab.at[pl.ds(rowbase, NUP)])
        return 0

    lax.fori_loop(0, 8, build_pair, 0)
    plsc.subcore_barrier()

    pltpu.sync_copy(inputs_hbm.at[pl.ds(wid * SPW, SPW)], in_v)

    def bidx(t, _):
        carry = jnp.float32(0)
        for k in range(4):
            sv = in_v[t, pl.ds(16 * k, 16)]
            sf = sv.astype(jnp.float32)
            incl = jnp.cumsum(sf)
            nu = (incl - sf + carry).astype(jnp.int32)
            carry = carry + jnp.sum(sf)
            idx_v[pl.ds(t * L + 16 * k, 16)] = (
                rowcore + sv * S_STRIDE + (iota + 16 * k) * I_STRIDE + nu)
        return 0

    lax.fori_loop(0, SPW, bidx, 0)

    def product(rows_v, off):
        def prod(j, accs):
            accs = list(accs)
            for r in range(8):
                row = off + 8 * j + r
                c = (r % 2) * 4
                for k in range(4):
                    accs[c + k] = accs[c + k] * rows_v[row, pl.ds(16 * k, 16)]
            return tuple(accs)

        ones = jnp.ones((16,), jnp.float32)
        accs = lax.fori_loop(0, L // 8, prod, (ones,) * 8)
        return (accs[0] * accs[4] + accs[1] * accs[5]
                + accs[2] * accs[6] + accs[3] * accs[7])

    def gather_pair(p, dst, sem):
        return pltpu.async_copy(
            tab.at[idx_v.at[pl.ds(p * PAIRW, PAIRW)]], dst, sem)

    def wait_pair(p, dst, sem):
        pltpu.make_async_copy(
            tab.at[idx_v.at[pl.ds(p * PAIRW, PAIRW)]], dst, sem).wait()

    gather_pair(0, rows0, sem0)

    def group(g, _):
        def quad(qq, _):
            p0 = g * 8 + 2 * qq
            s0 = 4 * qq
            gather_pair(p0 + 1, rows1, sem1)
            wait_pair(p0, rows0, sem0)
            tot_a = product(rows0, 0)
            tot_b = product(rows0, L)

            @pl.when(p0 < NPAIR - 2)
            def _():
                gather_pair(p0 + 2, rows0, sem0)

            wait_pair(p0 + 1, rows1, sem1)
            tot_c = product(rows1, 0)
            tot_d = product(rows1, L)
            tmp_v[pl.ds(s0 * 16, 16)] = tot_a
            tmp_v[pl.ds((s0 + 1) * 16, 16)] = tot_b
            tmp_v[pl.ds((s0 + 2) * 16, 16)] = tot_c
            tmp_v[pl.ds((s0 + 3) * 16, 16)] = tot_d
            return 0

        lax.fori_loop(0, 4, quad, 0)
        acc = jnp.zeros((16,), jnp.float32)
        for j in range(16):
            acc = acc + plsc.load_gather(tmp_v, [iota * 16 + j])
        out_v[pl.ds(g * 16, 16)] = acc
        return 0

    lax.fori_loop(0, GRP, group, 0)
    pltpu.sync_copy(out_v, out_hbm.at[pl.ds(wid * SPW, SPW)])


@jax.jit
def _seg_gps(eps_flat, inputs_i32):
    mesh = plsc.VectorSubcoreMesh(core_axis_name="c", subcore_axis_name="s")
    return pl.kernel(
        _sc_body,
        mesh=mesh,
        compiler_params=pltpu.CompilerParams(
            needs_layout_passes=False, use_tc_tiling_on_sc=False),
        out_type=jax.ShapeDtypeStruct((BATCH,), jnp.float32),
        scratch_types=[
            pltpu.HBM((_NC * TROWS, M), jnp.float32),
            pltpu.VMEM((SPW, L), jnp.int32),
            pltpu.VMEM((NUP * M,), jnp.int32),
            pltpu.VMEM((NUP, M), jnp.float32),
            pltpu.VMEM((SPW * L,), jnp.int32),
            pltpu.VMEM((PAIRW, M), jnp.float32),
            pltpu.VMEM((PAIRW, M), jnp.float32),
            pltpu.VMEM((256,), jnp.float32),
            pltpu.VMEM((SPW,), jnp.float32),
            pltpu.SemaphoreType.DMA,
            pltpu.SemaphoreType.DMA,
            pltpu.SemaphoreType.DMA,
        ],
    )(eps_flat, inputs_i32)


def kernel(inputs, epsilon):
    return _seg_gps(epsilon.reshape(-1), inputs.astype(jnp.int32))
